# baseline (speedup 1.0000x reference)
import functools

import jax
import jax.numpy as jnp
from jax import lax
from jax.experimental import pallas as pl
from jax.experimental.pallas import tpu as pltpu

_GEM_EPS = 1e-6
_BN_EPS = 1e-5
_ONE_THIRD = 1.0 / 3.0


def _pool_sums(x, ch=8):
    hw = x.shape[1]
    a1 = x[:, 0:ch, :]
    cq = jnp.maximum(a1, _GEM_EPS)
    a3 = cq * cq * cq
    for q in range(1, hw // ch):
        xq = x[:, q * ch : (q + 1) * ch, :]
        cq = jnp.maximum(xq, _GEM_EPS)
        a1 = a1 + xq
        a3 = a3 + cq * cq * cq
    return jnp.sum(a1, axis=1), jnp.sum(a3, axis=1)


def _fused_kernel(xh_ref, gamma_ref, beta_ref, xl_hbm, w_hbm,
                  cls_ref, bn_ref, gf_ref,
                  xl_buf, a1_scr, a3_scr, w_vmem, xl_sems, w_sem,
                  *, h_steps, l_steps, tch, c_h, slab,
                  inv_hw_l, inv_hw_h):
    j = pl.program_id(0)
    n_steps = h_steps + l_steps

    def _slab_copy(s):
        return pltpu.make_async_copy(
            xl_hbm.at[:, pl.ds(s * slab, slab), :],
            xl_buf.at[s % 2],
            xl_sems.at[s % 2],
        )

    @pl.when(j == 0)
    def _():
        _slab_copy(0).start()
        pltpu.make_async_copy(w_hbm, w_vmem, w_sem).start()

    @pl.when(j == h_steps - 1)
    def _():
        _slab_copy(1).start()

    @pl.when(j == h_steps + 1)
    def _():
        _slab_copy(2).start()

    @pl.when(j == h_steps + 2)
    def _():
        _slab_copy(3).start()

    @pl.when(j < h_steps)
    def _pool_hi():
        s1h, s3h = _pool_sums(xh_ref[...])
        gemh = jnp.exp(jnp.log(s3h * inv_hw_h) * _ONE_THIRD)
        gf_ref[:, pl.ds(j * tch, tch)] = gemh + s1h * inv_hw_h

    def _pool_low_step(s):
        _slab_copy(s).wait()
        p1, p3 = _pool_sums(xl_buf[s % 2])
        if s == 0:
            a1_scr[...] = p1
            a3_scr[...] = p3
        else:
            a1_scr[...] = a1_scr[...] + p1
            a3_scr[...] = a3_scr[...] + p3
        if s == l_steps - 1:
            s3l = a3_scr[...]
            geml = jnp.exp(jnp.log(s3l * inv_hw_l) * _ONE_THIRD)
            gf_ref[:, c_h:] = geml + a1_scr[...] * inv_hw_l

    for s in range(l_steps):
        @pl.when(j == h_steps + s)
        def _(s=s):
            _pool_low_step(s)

    @pl.when(j == n_steps)
    def _head():
        g = gf_ref[...]
        mean = jnp.mean(g, axis=0, keepdims=True)
        var = jnp.mean((g - mean) ** 2, axis=0, keepdims=True)
        y = (g - mean) * lax.rsqrt(var + _BN_EPS) * gamma_ref[...] + beta_ref[...]
        bn_ref[...] = y
        pltpu.make_async_copy(w_hbm, w_vmem, w_sem).wait()
        cls_ref[...] = jnp.dot(y, w_vmem[...],
                               preferred_element_type=jnp.float32)


def _fused_forward(x_low, x_hi, gamma, beta, w_t, *, h_steps=4, l_steps=4):
    n, hw_l, c_l = x_low.shape
    _, hw_h, c_h = x_hi.shape
    c = c_l + c_h
    k = w_t.shape[1]
    tch = c_h // h_steps
    slab = hw_l // l_steps
    steps = h_steps + l_steps + 1
    last_h = h_steps - 1

    return pl.pallas_call(
        functools.partial(_fused_kernel, h_steps=h_steps, l_steps=l_steps,
                          tch=tch, c_h=c_h, slab=slab,
                          inv_hw_l=1.0 / hw_l, inv_hw_h=1.0 / hw_h),
        out_shape=(
            jax.ShapeDtypeStruct((n, k), jnp.float32),
            jax.ShapeDtypeStruct((n, c), jnp.float32),
            jax.ShapeDtypeStruct((n, c), jnp.float32),
        ),
        grid=(steps,),
        in_specs=[
            pl.BlockSpec((n, hw_h, tch), lambda j: (0, 0, jnp.minimum(j, last_h))),
            pl.BlockSpec((1, c), lambda j: (0, 0)),
            pl.BlockSpec((1, c), lambda j: (0, 0)),
            pl.BlockSpec(memory_space=pl.ANY),
            pl.BlockSpec(memory_space=pl.ANY),
        ],
        out_specs=(
            pl.BlockSpec((n, k), lambda j: (0, 0)),
            pl.BlockSpec((n, c), lambda j: (0, 0)),
            pl.BlockSpec((n, c), lambda j: (0, 0)),
        ),
        scratch_shapes=[
            pltpu.VMEM((2, n, hw_l // l_steps, c_l), jnp.float32),
            pltpu.VMEM((n, c_l), jnp.float32),
            pltpu.VMEM((n, c_l), jnp.float32),
            pltpu.VMEM((c, k), jnp.float32),
            pltpu.SemaphoreType.DMA((2,)),
            pltpu.SemaphoreType.DMA,
        ],
        compiler_params=pltpu.CompilerParams(
            dimension_semantics=("arbitrary",)),
    )(x_hi, gamma, beta, x_low, w_t)


def kernel(featmap_low, featmap, gamma, beta, w_t):
    n, c_l, h_l, w_l = featmap_low.shape
    _, c_h, h_h, w_h = featmap.shape
    x_low = jnp.transpose(featmap_low, (0, 2, 3, 1)).reshape(n, h_l * w_l, c_l)
    x_hi = jnp.transpose(featmap, (0, 2, 3, 1)).reshape(n, h_h * w_h, c_h)
    return _fused_forward(x_low, x_hi, gamma, beta, w_t)


# speedup vs baseline: 1.0098x; 1.0098x over previous
import functools

import jax
import jax.numpy as jnp
from jax import lax
from jax.experimental import pallas as pl
from jax.experimental.pallas import tpu as pltpu

_GEM_EPS = 1e-6
_BN_EPS = 1e-5
_ONE_THIRD = 1.0 / 3.0


def _fused_kernel(xl_ref, xh_ref, gamma_ref, beta_ref, w_hbm_ref,
                  cls_ref, bn_ref, gf_ref, a1_scr, a3_scr, w_vmem, w_sem,
                  *, c_steps, tcl, tch, c_h, inv_hw_l, inv_hw_h):
    j = pl.program_id(0)
    p_steps = 2 * c_steps

    @pl.when(j == 0)
    def _start_w_copy():
        pltpu.make_async_copy(w_hbm_ref, w_vmem, w_sem).start()

    def _pool_sums(x, ch=8):
        hw = x.shape[1]
        a1 = x[:, 0:ch, :]
        cq = jnp.maximum(a1, _GEM_EPS)
        a3 = cq * cq * cq
        for q in range(1, hw // ch):
            xq = x[:, q * ch : (q + 1) * ch, :]
            cq = jnp.maximum(xq, _GEM_EPS)
            a1 = a1 + xq
            a3 = a3 + cq * cq * cq
        return jnp.sum(a1, axis=1), jnp.sum(a3, axis=1)

    @pl.when(j < c_steps)
    def _pool_first_half():
        lo = pl.ds(c_h + j * tcl, tcl)
        hi = pl.ds(j * tch, tch)
        s1l, s3l = _pool_sums(xl_ref[...])
        a1_scr[:, lo] = s1l
        a3_scr[:, lo] = s3l
        s1h, s3h = _pool_sums(xh_ref[...])
        a1_scr[:, hi] = s1h
        a3_scr[:, hi] = s3h

    @pl.when((j >= c_steps) & (j < p_steps))
    def _pool_second_half():
        ct = j - c_steps
        lo = pl.ds(c_h + ct * tcl, tcl)
        hi = pl.ds(ct * tch, tch)
        s1l, s3l = _pool_sums(xl_ref[...])
        s1l = s1l + a1_scr[:, lo]
        s3l = s3l + a3_scr[:, lo]
        geml = jnp.exp(jnp.log(s3l * inv_hw_l) * _ONE_THIRD)
        gf_ref[:, lo] = geml + s1l * inv_hw_l
        s1h, s3h = _pool_sums(xh_ref[...])
        s1h = s1h + a1_scr[:, hi]
        s3h = s3h + a3_scr[:, hi]
        gemh = jnp.exp(jnp.log(s3h * inv_hw_h) * _ONE_THIRD)
        gf_ref[:, hi] = gemh + s1h * inv_hw_h

    @pl.when(j == p_steps)
    def _head():
        g = gf_ref[...]
        mean = jnp.mean(g, axis=0, keepdims=True)
        var = jnp.mean((g - mean) ** 2, axis=0, keepdims=True)
        y = (g - mean) * lax.rsqrt(var + _BN_EPS) * gamma_ref[...] + beta_ref[...]
        bn_ref[...] = y
        pltpu.make_async_copy(w_hbm_ref, w_vmem, w_sem).wait()
        cls_ref[...] = jnp.dot(y, w_vmem[...],
                               preferred_element_type=jnp.float32)


def _fused_forward(x_low, x_hi, gamma, beta, w_t, *, c_steps=4):
    n, hw_l, c_l = x_low.shape
    _, hw_h, c_h = x_hi.shape
    c = c_l + c_h
    k = w_t.shape[1]
    tcl = c_l // c_steps
    tch = c_h // c_steps
    p_steps = 2 * c_steps
    steps = p_steps + 1
    last = p_steps - 1

    def _tile_idx(j):
        jc = jnp.minimum(j, last)
        return (0, jc // c_steps, jc % c_steps)

    return pl.pallas_call(
        functools.partial(_fused_kernel, c_steps=c_steps, tcl=tcl, tch=tch,
                          c_h=c_h, inv_hw_l=1.0 / hw_l, inv_hw_h=1.0 / hw_h),
        out_shape=(
            jax.ShapeDtypeStruct((n, k), jnp.float32),
            jax.ShapeDtypeStruct((n, c), jnp.float32),
            jax.ShapeDtypeStruct((n, c), jnp.float32),
        ),
        grid=(steps,),
        in_specs=[
            pl.BlockSpec((n, hw_l // 2, tcl), _tile_idx),
            pl.BlockSpec((n, hw_h // 2, tch), _tile_idx),
            pl.BlockSpec((1, c), lambda j: (0, 0)),
            pl.BlockSpec((1, c), lambda j: (0, 0)),
            pl.BlockSpec(memory_space=pl.ANY),
        ],
        out_specs=(
            pl.BlockSpec((n, k), lambda j: (0, 0)),
            pl.BlockSpec((n, c), lambda j: (0, 0)),
            pl.BlockSpec((n, c), lambda j: (0, 0)),
        ),
        scratch_shapes=[
            pltpu.VMEM((n, c), jnp.float32),
            pltpu.VMEM((n, c), jnp.float32),
            pltpu.VMEM((c, k), jnp.float32),
            pltpu.SemaphoreType.DMA,
        ],
        compiler_params=pltpu.CompilerParams(
            dimension_semantics=("arbitrary",)),
    )(x_low, x_hi, gamma, beta, w_t)


def kernel(featmap_low, featmap, gamma, beta, w_t):
    n, c_l, h_l, w_l = featmap_low.shape
    _, c_h, h_h, w_h = featmap.shape
    x_low = jnp.transpose(featmap_low, (0, 2, 3, 1)).reshape(n, h_l * w_l, c_l)
    x_hi = jnp.transpose(featmap, (0, 2, 3, 1)).reshape(n, h_h * w_h, c_h)
    return _fused_forward(x_low, x_hi, gamma, beta, w_t)


# speedup vs baseline: 1.0196x; 1.0098x over previous
import functools

import jax
import jax.numpy as jnp
from jax import lax
from jax.experimental import pallas as pl
from jax.experimental.pallas import tpu as pltpu

_GEM_EPS = 1e-6
_BN_EPS = 1e-5
_ONE_THIRD = 1.0 / 3.0


def _pool_sums_2d(x, ch=8):
    hw = x.shape[0]
    a1 = x[0:ch, :]
    cq = jnp.maximum(a1, _GEM_EPS)
    a3 = cq * cq * cq
    for q in range(1, hw // ch):
        xq = x[q * ch : (q + 1) * ch, :]
        cq = jnp.maximum(xq, _GEM_EPS)
        a1 = a1 + xq
        a3 = a3 + cq * cq * cq
    return (jnp.sum(a1, axis=0, keepdims=True),
            jnp.sum(a3, axis=0, keepdims=True))


def _fused_kernel(xl_ref, xh_ref, gamma_ref, beta_ref, w_hbm_ref,
                  cls_ref, bn_ref, gf_ref, w_vmem, w_sem,
                  *, p_steps, n_per, hw_l, hw_h, c_h,
                  inv_hw_l, inv_hw_h):
    j = pl.program_id(0)

    @pl.when(j == 0)
    def _start_w_copy():
        pltpu.make_async_copy(w_hbm_ref, w_vmem, w_sem).start()

    def _pool_map(x_ref, hw, inv_hw):
        rows = []
        for i in range(n_per):
            s1, s3 = _pool_sums_2d(x_ref[i * hw : (i + 1) * hw, :])
            gem = jnp.exp(jnp.log(s3 * inv_hw) * _ONE_THIRD)
            rows.append(gem + s1 * inv_hw)
        return jnp.concatenate(rows, axis=0)

    @pl.when(j < p_steps)
    def _pool():
        nrow = pl.ds(j * n_per, n_per)
        gf_ref[nrow, c_h:] = _pool_map(xl_ref, hw_l, inv_hw_l)
        gf_ref[nrow, :c_h] = _pool_map(xh_ref, hw_h, inv_hw_h)

    @pl.when(j == p_steps)
    def _head():
        g = gf_ref[...]
        mean = jnp.mean(g, axis=0, keepdims=True)
        var = jnp.mean((g - mean) ** 2, axis=0, keepdims=True)
        y = (g - mean) * lax.rsqrt(var + _BN_EPS) * gamma_ref[...] + beta_ref[...]
        bn_ref[...] = y
        pltpu.make_async_copy(w_hbm_ref, w_vmem, w_sem).wait()
        cls_ref[...] = jnp.dot(y, w_vmem[...],
                               preferred_element_type=jnp.float32)


def _fused_forward(x_low, x_hi, gamma, beta, w_t, *, n_per=8):
    n, hw_l, c_l = x_low.shape
    _, hw_h, c_h = x_hi.shape
    c = c_l + c_h
    k = w_t.shape[1]
    p_steps = n // n_per
    steps = p_steps + 1
    last = p_steps - 1
    xl2 = x_low.reshape(n * hw_l, c_l)
    xh2 = x_hi.reshape(n * hw_h, c_h)

    return pl.pallas_call(
        functools.partial(_fused_kernel, p_steps=p_steps, n_per=n_per,
                          hw_l=hw_l, hw_h=hw_h, c_h=c_h,
                          inv_hw_l=1.0 / hw_l, inv_hw_h=1.0 / hw_h),
        out_shape=(
            jax.ShapeDtypeStruct((n, k), jnp.float32),
            jax.ShapeDtypeStruct((n, c), jnp.float32),
            jax.ShapeDtypeStruct((n, c), jnp.float32),
        ),
        grid=(steps,),
        in_specs=[
            pl.BlockSpec((n_per * hw_l, c_l), lambda j: (jnp.minimum(j, last), 0)),
            pl.BlockSpec((n_per * hw_h, c_h), lambda j: (jnp.minimum(j, last), 0)),
            pl.BlockSpec((1, c), lambda j: (0, 0)),
            pl.BlockSpec((1, c), lambda j: (0, 0)),
            pl.BlockSpec(memory_space=pl.ANY),
        ],
        out_specs=(
            pl.BlockSpec((n, k), lambda j: (0, 0)),
            pl.BlockSpec((n, c), lambda j: (0, 0)),
            pl.BlockSpec((n, c), lambda j: (0, 0)),
        ),
        scratch_shapes=[
            pltpu.VMEM((c, k), jnp.float32),
            pltpu.SemaphoreType.DMA,
        ],
        compiler_params=pltpu.CompilerParams(
            dimension_semantics=("arbitrary",)),
    )(xl2, xh2, gamma, beta, w_t)


def kernel(featmap_low, featmap, gamma, beta, w_t):
    n, c_l, h_l, w_l = featmap_low.shape
    _, c_h, h_h, w_h = featmap.shape
    x_low = jnp.transpose(featmap_low, (0, 2, 3, 1)).reshape(n, h_l * w_l, c_l)
    x_hi = jnp.transpose(featmap, (0, 2, 3, 1)).reshape(n, h_h * w_h, c_h)
    return _fused_forward(x_low, x_hi, gamma, beta, w_t)


# speedup vs baseline: 1.0864x; 1.0654x over previous
import functools

import jax
import jax.numpy as jnp
from jax import lax
from jax.experimental import pallas as pl
from jax.experimental.pallas import tpu as pltpu

_GEM_EPS = 1e-6
_BN_EPS = 1e-5
_ONE_THIRD = 1.0 / 3.0


def _fused_kernel(xl_ref, xh_ref, gamma_ref, beta_ref, w_hbm_ref,
                  cls_ref, bn_ref, gf_ref, w_vmem, w_sem,
                  *, p_steps, tcl, tch, c_h, inv_hw_l, inv_hw_h):
    j = pl.program_id(0)

    @pl.when(j == 0)
    def _start_w_copy():
        pltpu.make_async_copy(w_hbm_ref, w_vmem, w_sem).start()

    def _pool_sums(x, ch=8):
        hw = x.shape[1]
        a1 = x[:, 0:ch, :]
        cq = jnp.maximum(a1, _GEM_EPS)
        a3 = cq * cq * cq
        for q in range(1, hw // ch):
            xq = x[:, q * ch : (q + 1) * ch, :]
            cq = jnp.maximum(xq, _GEM_EPS)
            a1 = a1 + xq
            a3 = a3 + cq * cq * cq
        return jnp.sum(a1, axis=1), jnp.sum(a3, axis=1)

    @pl.when(j < p_steps)
    def _pool():
        s1l, s3l = _pool_sums(xl_ref[...])
        geml = jnp.exp(jnp.log(s3l * inv_hw_l) * _ONE_THIRD)
        gf_ref[:, pl.ds(c_h + j * tcl, tcl)] = geml + s1l * inv_hw_l

        s1h, s3h = _pool_sums(xh_ref[...])
        gemh = jnp.exp(jnp.log(s3h * inv_hw_h) * _ONE_THIRD)
        gf_ref[:, pl.ds(j * tch, tch)] = gemh + s1h * inv_hw_h

    @pl.when(j == p_steps)
    def _head():
        g = gf_ref[...]
        mean = jnp.mean(g, axis=0, keepdims=True)
        var = jnp.mean((g - mean) ** 2, axis=0, keepdims=True)
        y = (g - mean) * lax.rsqrt(var + _BN_EPS) * gamma_ref[...] + beta_ref[...]
        bn_ref[...] = y
        pltpu.make_async_copy(w_hbm_ref, w_vmem, w_sem).wait()
        cls_ref[...] = jnp.dot(y, w_vmem[...],
                               preferred_element_type=jnp.float32)


def _fused_forward(x_low, x_hi, gamma, beta, w_t, *, p_steps=4):
    n, hw_l, c_l = x_low.shape
    _, hw_h, c_h = x_hi.shape
    c = c_l + c_h
    k = w_t.shape[1]
    tcl = c_l // p_steps
    tch = c_h // p_steps
    steps = p_steps + 1
    last = p_steps - 1

    return pl.pallas_call(
        functools.partial(_fused_kernel, p_steps=p_steps, tcl=tcl, tch=tch,
                          c_h=c_h, inv_hw_l=1.0 / hw_l, inv_hw_h=1.0 / hw_h),
        out_shape=(
            jax.ShapeDtypeStruct((n, k), jnp.float32),
            jax.ShapeDtypeStruct((n, c), jnp.float32),
            jax.ShapeDtypeStruct((n, c), jnp.float32),
        ),
        grid=(steps,),
        in_specs=[
            pl.BlockSpec((n, hw_l, tcl), lambda j: (0, 0, jnp.minimum(j, last))),
            pl.BlockSpec((n, hw_h, tch), lambda j: (0, 0, jnp.minimum(j, last))),
            pl.BlockSpec((1, c), lambda j: (0, 0)),
            pl.BlockSpec((1, c), lambda j: (0, 0)),
            pl.BlockSpec(memory_space=pl.ANY),
        ],
        out_specs=(
            pl.BlockSpec((n, k), lambda j: (0, 0)),
            pl.BlockSpec((n, c), lambda j: (0, 0)),
            pl.BlockSpec((n, c), lambda j: (0, 0)),
        ),
        scratch_shapes=[
            pltpu.VMEM((c, k), jnp.float32),
            pltpu.SemaphoreType.DMA,
        ],
        compiler_params=pltpu.CompilerParams(
            dimension_semantics=("arbitrary",)),
    )(x_low, x_hi, gamma, beta, w_t)


def kernel(featmap_low, featmap, gamma, beta, w_t):
    n, c_l, h_l, w_l = featmap_low.shape
    _, c_h, h_h, w_h = featmap.shape
    x_low = jnp.transpose(featmap_low, (0, 2, 3, 1)).reshape(n, h_l * w_l, c_l)
    x_hi = jnp.transpose(featmap, (0, 2, 3, 1)).reshape(n, h_h * w_h, c_h)
    return _fused_forward(x_low, x_hi, gamma, beta, w_t)
